# baseline (speedup 1.0000x reference)
import dataclasses
import functools

import jax
import jax.numpy as jnp
from jax import lax
from jax.experimental import pallas as pl
from jax.experimental.pallas import tpu as pltpu
from jax.experimental.pallas import tpu_sc as plsc

_IN_CH = 128
_OUT_CH = 64
_STEPS = 5
_PTS = 16
_LANES = 16
_NW = 32
_MAGIC = 12582912.0


def _tab_body(ang_ref, ctab_ref, stab_ref):
    a = ang_ref[...]
    ctab_ref[...] = jnp.cos(a) / _STEPS
    stab_ref[...] = jnp.sin(a) / _STEPS


def _sc_body(data_hbm, ctab_hbm, stab_hbm, out_hbm,
             xbuf, obuf, ctab_v, stab_v):
    wid = lax.axis_index("s") * 2 + lax.axis_index("c")
    rows = data_hbm.shape[0] // _NW
    base = wid * rows
    pltpu.sync_copy(data_hbm.at[pl.ds(base, rows)], xbuf)
    pltpu.sync_copy(ctab_hbm, ctab_v)
    pltpu.sync_copy(stab_hbm, stab_v)

    @pl.loop(0, rows)
    def _row(r):
        accs = []
        for ci in range(_IN_CH // _LANES):
            x = xbuf[r, pl.ds(ci * _LANES, _LANES)]
            for ix in range(_STEPS):
                t = x * (_PTS / 2.0) + (_PTS / 2.0 + _PTS * ix + _MAGIC)
                p = t - _MAGIC
                p = jnp.minimum(jnp.maximum(p, float(_PTS * ix)),
                                float(_PTS * ix + _PTS - 1))
                idx = p.astype(jnp.int32)
                c = plsc.load_gather(ctab_v, [idx])
                s = plsc.load_gather(stab_v, [idx])
                x = x + (c + x * s)
            accs.append(x)
        while len(accs) > 1:
            accs = [a + b for a, b in zip(accs[::2], accs[1::2])]
        rsum = jnp.sum(accs[0])
        splat = jnp.full((_LANES,), rsum, jnp.float32)
        for j in range(_OUT_CH // _LANES):
            obuf[r, pl.ds(j * _LANES, _LANES)] = splat

    pltpu.sync_copy(obuf, out_hbm.at[pl.ds(base, rows)])


def kernel(data, angles):
    b = data.shape[0]
    ctab, stab = pl.pallas_call(
        _tab_body,
        out_shape=(jax.ShapeDtypeStruct((_STEPS, _PTS), jnp.float32),
                   jax.ShapeDtypeStruct((_STEPS, _PTS), jnp.float32)),
    )(angles)
    ctab = ctab.reshape(_STEPS * _PTS)
    stab = stab.reshape(_STEPS * _PTS)

    rows = b // _NW
    cp = pltpu.CompilerParams()
    if "needs_layout_passes" in pltpu.CompilerParams.__dataclass_fields__:
        cp = dataclasses.replace(cp, needs_layout_passes=False)
    sc_call = pl.kernel(
        _sc_body,
        out_type=jax.ShapeDtypeStruct((b, _OUT_CH), jnp.float32),
        mesh=plsc.VectorSubcoreMesh(core_axis_name="c", subcore_axis_name="s"),
        scratch_types=[
            pltpu.VMEM((rows, _IN_CH), jnp.float32),
            pltpu.VMEM((rows, _OUT_CH), jnp.float32),
            pltpu.VMEM((_STEPS * _PTS,), jnp.float32),
            pltpu.VMEM((_STEPS * _PTS,), jnp.float32),
        ],
        compiler_params=cp,
    )
    return sc_call(data, ctab, stab)


# speedup vs baseline: 1.0389x; 1.0389x over previous
import dataclasses

import jax
import jax.numpy as jnp
from jax import lax
from jax.experimental import pallas as pl
from jax.experimental.pallas import tpu as pltpu
from jax.experimental.pallas import tpu_sc as plsc

_IN_CH = 128
_OUT_CH = 64
_STEPS = 5
_PTS = 16
_LANES = 16
_NW = 32
_MAGIC = 12582912.0


def _sc_body(data_hbm, ang_hbm, out_hbm, xbuf, obuf, ang_v, ctab_v, btab_v):
    wid = lax.axis_index("s") * 2 + lax.axis_index("c")
    rows = data_hbm.shape[0] // _NW
    base = wid * rows
    pltpu.sync_copy(data_hbm.at[pl.ds(base, rows)], xbuf)
    pltpu.sync_copy(ang_hbm, ang_v)

    for i in range(_STEPS):
        a = ang_v[i, :]
        a2 = a * a
        c = 1.0 / 201600.0
        for k in (-1.0 / 3600.0, 1.0 / 120.0, -1.0 / 10.0, 1.0 / 5.0):
            c = c * a2 + k
        ctab_v[pl.ds(i * _PTS, _PTS)] = c
        sp = 1.0 / 1814400.0
        for k in (-1.0 / 25200.0, 1.0 / 600.0, -1.0 / 30.0, 1.0 / 5.0):
            sp = sp * a2 + k
        btab_v[pl.ds(i * _PTS, _PTS)] = a * sp + 1.0

    @pl.loop(0, rows)
    def _row(r):
        accs = []
        for ci in range(_IN_CH // _LANES):
            x = xbuf[r, pl.ds(ci * _LANES, _LANES)]
            for ix in range(_STEPS):
                t = x * (_PTS / 2.0) + (_PTS / 2.0 + _PTS * ix + _MAGIC)
                p = t - _MAGIC
                p = jnp.minimum(jnp.maximum(p, float(_PTS * ix)),
                                float(_PTS * ix + _PTS - 1))
                idx = p.astype(jnp.int32)
                c = plsc.load_gather(ctab_v, [idx])
                b = plsc.load_gather(btab_v, [idx])
                x = c + x * b
            accs.append(x)
        while len(accs) > 1:
            accs = [u + v for u, v in zip(accs[::2], accs[1::2])]
        rsum = jnp.sum(accs[0])
        splat = jnp.full((_LANES,), rsum, jnp.float32)
        for j in range(_OUT_CH // _LANES):
            obuf[r, pl.ds(j * _LANES, _LANES)] = splat

    pltpu.sync_copy(obuf, out_hbm.at[pl.ds(base, rows)])


def kernel(data, angles):
    b = data.shape[0]
    rows = b // _NW
    cp = pltpu.CompilerParams()
    if "needs_layout_passes" in pltpu.CompilerParams.__dataclass_fields__:
        cp = dataclasses.replace(cp, needs_layout_passes=False)
    sc_call = pl.kernel(
        _sc_body,
        out_type=jax.ShapeDtypeStruct((b, _OUT_CH), jnp.float32),
        mesh=plsc.VectorSubcoreMesh(core_axis_name="c", subcore_axis_name="s"),
        scratch_types=[
            pltpu.VMEM((rows, _IN_CH), jnp.float32),
            pltpu.VMEM((rows, _OUT_CH), jnp.float32),
            pltpu.VMEM((_STEPS, _PTS), jnp.float32),
            pltpu.VMEM((_STEPS * _PTS,), jnp.float32),
            pltpu.VMEM((_STEPS * _PTS,), jnp.float32),
        ],
        compiler_params=cp,
    )
    return sc_call(data, angles)


# speedup vs baseline: 1.1213x; 1.0793x over previous
import dataclasses

import jax
import jax.numpy as jnp
from jax import lax
from jax.experimental import pallas as pl
from jax.experimental.pallas import tpu as pltpu
from jax.experimental.pallas import tpu_sc as plsc

_IN_CH = 128
_OUT_CH = 64
_STEPS = 5
_PTS = 16
_LANES = 16
_NW = 32
_MAGIC = 12582912.0


def _sc_body(data_hbm, ang_hbm, out_hbm, xbuf, obuf, ang_v, ctab_v, btab_v):
    wid = lax.axis_index("s") * 2 + lax.axis_index("c")
    rows = data_hbm.shape[0] // _NW
    base = wid * rows
    pltpu.sync_copy(data_hbm.at[pl.ds(base, rows)], xbuf)
    pltpu.sync_copy(ang_hbm, ang_v)

    for i in range(_STEPS):
        a = ang_v[i, :]
        a2 = a * a
        c = 1.0 / 201600.0
        for k in (-1.0 / 3600.0, 1.0 / 120.0, -1.0 / 10.0, 1.0 / 5.0):
            c = c * a2 + k
        ctab_v[pl.ds(i * _PTS, _PTS)] = c
        sp = 1.0 / 1814400.0
        for k in (-1.0 / 25200.0, 1.0 / 600.0, -1.0 / 30.0, 1.0 / 5.0):
            sp = sp * a2 + k
        btab_v[pl.ds(i * _PTS, _PTS)] = a * sp + 1.0

    @pl.loop(0, rows)
    def _row(r):
        accs = []
        for ci in range(_IN_CH // _LANES):
            x = xbuf[r, pl.ds(ci * _LANES, _LANES)]
            for ix in range(_STEPS):
                t = x * (_PTS / 2.0) + (_PTS / 2.0 + _PTS * ix + _MAGIC)
                p = t - _MAGIC
                p = jnp.minimum(jnp.maximum(p, float(_PTS * ix)),
                                float(_PTS * ix + _PTS - 1))
                idx = p.astype(jnp.int32)
                c = plsc.load_gather(ctab_v, [idx])
                b = plsc.load_gather(btab_v, [idx])
                x = c + x * b
            accs.append(x)
        while len(accs) > 1:
            accs = [u + v for u, v in zip(accs[::2], accs[1::2])]
        rsum = jnp.sum(accs[0])
        splat = jnp.full((_LANES,), rsum, jnp.float32)
        for j in range(_OUT_CH // _LANES):
            obuf[r, pl.ds(j * _LANES, _LANES)] = splat

    pltpu.sync_copy(obuf, out_hbm.at[pl.ds(base, rows)])


def _tc_body(data_ref, ang_ref, out_ref):
    a = ang_ref[...]
    ctab = jnp.cos(a) / _STEPS
    stab = jnp.sin(a) / _STEPS
    x = data_ref[...]
    for ix in range(_STEPS):
        z = (1.0 + x) * (_PTS / 2.0)
        posf = jnp.clip(jnp.round(z), 0.0, float(_PTS - 1))
        pos = posf.astype(jnp.int32)
        cb = jnp.broadcast_to(ctab[ix][None, :], (x.shape[0], _PTS))
        sb = jnp.broadcast_to(stab[ix][None, :], (x.shape[0], _PTS))
        c = jnp.take_along_axis(cb, pos, axis=1)
        s = jnp.take_along_axis(sb, pos, axis=1)
        x = x + (c + x * s)
    r = jnp.sum(x, axis=1, keepdims=True)
    out_ref[...] = jnp.broadcast_to(r, (x.shape[0], _OUT_CH))


def kernel(data, angles):
    b = data.shape[0]
    b_sc = b // 2
    cp = pltpu.CompilerParams()
    if "needs_layout_passes" in pltpu.CompilerParams.__dataclass_fields__:
        cp = dataclasses.replace(cp, needs_layout_passes=False)
    sc_call = pl.kernel(
        _sc_body,
        out_type=jax.ShapeDtypeStruct((b_sc, _OUT_CH), jnp.float32),
        mesh=plsc.VectorSubcoreMesh(core_axis_name="c", subcore_axis_name="s"),
        scratch_types=[
            pltpu.VMEM((b_sc // _NW, _IN_CH), jnp.float32),
            pltpu.VMEM((b_sc // _NW, _OUT_CH), jnp.float32),
            pltpu.VMEM((_STEPS, _PTS), jnp.float32),
            pltpu.VMEM((_STEPS * _PTS,), jnp.float32),
            pltpu.VMEM((_STEPS * _PTS,), jnp.float32),
        ],
        compiler_params=cp,
    )
    out_sc = sc_call(data[:b_sc], angles)
    out_tc = pl.pallas_call(
        _tc_body,
        out_shape=jax.ShapeDtypeStruct((b - b_sc, _OUT_CH), jnp.float32),
    )(data[b_sc:], angles)
    return jnp.concatenate([out_sc, out_tc], axis=0)


# speedup vs baseline: 1.1910x; 1.0621x over previous
import dataclasses

import jax
import jax.numpy as jnp
from jax import lax
from jax.experimental import pallas as pl
from jax.experimental.pallas import tpu as pltpu
from jax.experimental.pallas import tpu_sc as plsc

_IN_CH = 128
_OUT_CH = 64
_STEPS = 5
_PTS = 16
_LANES = 16
_NW = 32
_MAGIC = 12582912.0


def _sc_body(data_hbm, ang_hbm, out_hbm, xbuf, obuf, ang_v, ctab_v, btab_v):
    wid = lax.axis_index("s") * 2 + lax.axis_index("c")
    rows = out_hbm.shape[0] // _NW
    base = wid * rows
    pltpu.sync_copy(data_hbm.at[pl.ds(base, rows)], xbuf)
    pltpu.sync_copy(ang_hbm, ang_v)

    for i in range(_STEPS):
        a = ang_v[i, :]
        a2 = a * a
        c = 1.0 / 201600.0
        for k in (-1.0 / 3600.0, 1.0 / 120.0, -1.0 / 10.0, 1.0 / 5.0):
            c = c * a2 + k
        ctab_v[pl.ds(i * _PTS, _PTS)] = c
        sp = 1.0 / 1814400.0
        for k in (-1.0 / 25200.0, 1.0 / 600.0, -1.0 / 30.0, 1.0 / 5.0):
            sp = sp * a2 + k
        btab_v[pl.ds(i * _PTS, _PTS)] = a * sp + 1.0

    @pl.loop(0, rows)
    def _row(r):
        accs = []
        for ci in range(_IN_CH // _LANES):
            x = xbuf[r, pl.ds(ci * _LANES, _LANES)]
            for ix in range(_STEPS):
                t = x * (_PTS / 2.0) + (_PTS / 2.0 + _PTS * ix + _MAGIC)
                p = t - _MAGIC
                p = jnp.minimum(jnp.maximum(p, float(_PTS * ix)),
                                float(_PTS * ix + _PTS - 1))
                idx = p.astype(jnp.int32)
                c = plsc.load_gather(ctab_v, [idx])
                b = plsc.load_gather(btab_v, [idx])
                x = c + x * b
            accs.append(x)
        while len(accs) > 1:
            accs = [u + v for u, v in zip(accs[::2], accs[1::2])]
        rsum = jnp.sum(accs[0])
        splat = jnp.full((_LANES,), rsum, jnp.float32)
        for j in range(_OUT_CH // _LANES):
            obuf[r, pl.ds(j * _LANES, _LANES)] = splat

    pltpu.sync_copy(obuf, out_hbm.at[pl.ds(base, rows)])


def _tc_body(data_ref, ang_ref, out_ref):
    a = ang_ref[...]
    ctab = jnp.cos(a) / _STEPS
    stab = jnp.sin(a) / _STEPS
    skip = data_ref.shape[0] - out_ref.shape[0]
    x = data_ref[pl.ds(skip, out_ref.shape[0]), :]
    for ix in range(_STEPS):
        z = (1.0 + x) * (_PTS / 2.0)
        posf = jnp.clip(jnp.round(z), 0.0, float(_PTS - 1))
        pos = posf.astype(jnp.int32)
        cb = jnp.broadcast_to(ctab[ix][None, :], (x.shape[0], _PTS))
        sb = jnp.broadcast_to(stab[ix][None, :], (x.shape[0], _PTS))
        c = jnp.take_along_axis(cb, pos, axis=1)
        s = jnp.take_along_axis(sb, pos, axis=1)
        x = x + (c + x * s)
    r = jnp.sum(x, axis=1, keepdims=True)
    out_ref[...] = jnp.broadcast_to(r, (x.shape[0], _OUT_CH))


def kernel(data, angles):
    b = data.shape[0]
    b_sc = b // 4
    cp = pltpu.CompilerParams()
    if "needs_layout_passes" in pltpu.CompilerParams.__dataclass_fields__:
        cp = dataclasses.replace(cp, needs_layout_passes=False)
    sc_call = pl.kernel(
        _sc_body,
        out_type=jax.ShapeDtypeStruct((b_sc, _OUT_CH), jnp.float32),
        mesh=plsc.VectorSubcoreMesh(core_axis_name="c", subcore_axis_name="s"),
        scratch_types=[
            pltpu.VMEM((b_sc // _NW, _IN_CH), jnp.float32),
            pltpu.VMEM((b_sc // _NW, _OUT_CH), jnp.float32),
            pltpu.VMEM((_STEPS, _PTS), jnp.float32),
            pltpu.VMEM((_STEPS * _PTS,), jnp.float32),
            pltpu.VMEM((_STEPS * _PTS,), jnp.float32),
        ],
        compiler_params=cp,
    )
    out_sc = sc_call(data, angles)
    out_tc = pl.pallas_call(
        _tc_body,
        out_shape=jax.ShapeDtypeStruct((b - b_sc, _OUT_CH), jnp.float32),
    )(data, angles)
    return jnp.concatenate([out_sc, out_tc], axis=0)


# speedup vs baseline: 1.2143x; 1.0196x over previous
import dataclasses

import jax
import jax.numpy as jnp
from jax import lax
from jax.experimental import pallas as pl
from jax.experimental.pallas import tpu as pltpu
from jax.experimental.pallas import tpu_sc as plsc

_IN_CH = 128
_OUT_CH = 64
_STEPS = 5
_PTS = 16
_LANES = 16
_NW = 32
_MAGIC = 12582912.0


def _sc_body(data_hbm, ang_hbm, out_hbm, xbuf, obuf, ang_v, ctab_v, btab_v,
             dsem):
    wid = lax.axis_index("s") * 2 + lax.axis_index("c")
    rows = out_hbm.shape[0] // _NW
    base = wid * rows
    dcp = pltpu.async_copy(data_hbm.at[pl.ds(base, rows)], xbuf, dsem)
    pltpu.sync_copy(ang_hbm, ang_v)

    for i in range(_STEPS):
        a = ang_v[i, :]
        a2 = a * a
        c = 1.0 / 201600.0
        for k in (-1.0 / 3600.0, 1.0 / 120.0, -1.0 / 10.0, 1.0 / 5.0):
            c = c * a2 + k
        ctab_v[pl.ds(i * _PTS, _PTS)] = c
        sp = 1.0 / 1814400.0
        for k in (-1.0 / 25200.0, 1.0 / 600.0, -1.0 / 30.0, 1.0 / 5.0):
            sp = sp * a2 + k
        btab_v[pl.ds(i * _PTS, _PTS)] = a * sp + 1.0
    dcp.wait()

    @pl.loop(0, rows)
    def _row(r):
        accs = []
        for ci in range(_IN_CH // _LANES):
            x = xbuf[r, pl.ds(ci * _LANES, _LANES)]
            for ix in range(_STEPS):
                t = x * (_PTS / 2.0) + (_PTS / 2.0 + _PTS * ix + _MAGIC)
                p = t - _MAGIC
                p = jnp.minimum(jnp.maximum(p, float(_PTS * ix)),
                                float(_PTS * ix + _PTS - 1))
                idx = p.astype(jnp.int32)
                c = plsc.load_gather(ctab_v, [idx])
                b = plsc.load_gather(btab_v, [idx])
                x = c + x * b
            accs.append(x)
        while len(accs) > 1:
            accs = [u + v for u, v in zip(accs[::2], accs[1::2])]
        rsum = jnp.sum(accs[0])
        splat = jnp.full((_LANES,), rsum, jnp.float32)
        for j in range(_OUT_CH // _LANES):
            obuf[r, pl.ds(j * _LANES, _LANES)] = splat

    pltpu.sync_copy(obuf, out_hbm.at[pl.ds(base, rows)])


def _tc_body(data_ref, ang_ref, out_ref):
    a = ang_ref[...]
    ctab = jnp.cos(a) / _STEPS
    stab = jnp.sin(a) / _STEPS
    skip = data_ref.shape[0] - out_ref.shape[0]
    x = data_ref[pl.ds(skip, out_ref.shape[0]), :]
    for ix in range(_STEPS):
        z = (1.0 + x) * (_PTS / 2.0)
        posf = jnp.clip(jnp.round(z), 0.0, float(_PTS - 1))
        pos = posf.astype(jnp.int32)
        cb = jnp.broadcast_to(ctab[ix][None, :], (x.shape[0], _PTS))
        sb = jnp.broadcast_to(stab[ix][None, :], (x.shape[0], _PTS))
        c = jnp.take_along_axis(cb, pos, axis=1)
        s = jnp.take_along_axis(sb, pos, axis=1)
        x = x + (c + x * s)
    r = jnp.sum(x, axis=1, keepdims=True)
    out_ref[...] = jnp.broadcast_to(r, (x.shape[0], _OUT_CH))


def kernel(data, angles):
    b = data.shape[0]
    b_sc = b // 4
    cp = pltpu.CompilerParams()
    if "needs_layout_passes" in pltpu.CompilerParams.__dataclass_fields__:
        cp = dataclasses.replace(cp, needs_layout_passes=False)
    sc_call = pl.kernel(
        _sc_body,
        out_type=jax.ShapeDtypeStruct((b_sc, _OUT_CH), jnp.float32),
        mesh=plsc.VectorSubcoreMesh(core_axis_name="c", subcore_axis_name="s"),
        scratch_types=[
            pltpu.VMEM((b_sc // _NW, _IN_CH), jnp.float32),
            pltpu.VMEM((b_sc // _NW, _OUT_CH), jnp.float32),
            pltpu.VMEM((_STEPS, _PTS), jnp.float32),
            pltpu.VMEM((_STEPS * _PTS,), jnp.float32),
            pltpu.VMEM((_STEPS * _PTS,), jnp.float32),
            pltpu.SemaphoreType.DMA,
        ],
        compiler_params=cp,
    )
    out_sc = sc_call(data, angles)
    out_tc = pl.pallas_call(
        _tc_body,
        out_shape=jax.ShapeDtypeStruct((b - b_sc, _OUT_CH), jnp.float32),
    )(data, angles)
    return jnp.concatenate([out_sc, out_tc], axis=0)
